# baseline (speedup 1.0000x reference)
import functools

import jax
import jax.numpy as jnp
from jax import lax
from jax.experimental import pallas as pl
from jax.experimental.pallas import tpu as pltpu
from jax.experimental.pallas import tpu_sc as plsc

B, N, D = 4, 50000, 128
K = 25000


_SBLK = 2000


def _score_body(x_ref, w_ref, b_ref, o_ref):
    xb = x_ref[0].astype(jnp.bfloat16)
    wb = w_ref[...].astype(jnp.bfloat16)
    logits = jnp.dot(xb, wb, preferred_element_type=jnp.float32)
    o_ref[0, 0, :] = jax.nn.sigmoid(logits[:, 0] + b_ref[0, 0])


def _scores(x, W, b):
    nblk = N // _SBLK
    grid = (B, nblk)
    out = pl.pallas_call(
        _score_body,
        grid=grid,
        in_specs=[
            pl.BlockSpec((1, _SBLK, D), lambda i, j: (i, j, 0)),
            pl.BlockSpec((D, D), lambda i, j: (0, 0)),
            pl.BlockSpec(memory_space=pltpu.SMEM),
        ],
        out_specs=pl.BlockSpec((1, 1, _SBLK), lambda i, j: (i * nblk + j, 0, 0)),
        out_shape=jax.ShapeDtypeStruct((B * nblk, 1, _SBLK), jnp.float32),
    )(x, jnp.zeros((D, D), jnp.float32).at[:, 0].set(W[0]), b.reshape(1, 1))
    return out.reshape(B, N)


_SR, _SC_ = 512, 128
_NPAD = _SR * _SC_


def _sort_body(k_ref, ko_ref, io_ref):
    k = k_ref[0]
    row = lax.broadcasted_iota(jnp.int32, (_SR, _SC_), 0)
    lane = lax.broadcasted_iota(jnp.int32, (_SR, _SC_), 1)
    n = row * _SC_ + lane
    i = n
    def _partner(v, lsh, rsh):
        return pltpu.roll(pltpu.roll(v, lsh, 1), rsh, 0)

    for lev in range(1, 17):
        kb = 1 << lev
        up = (n & kb) == 0

        def step(t, carry, lev=lev, up=up):
            k, i = carry
            s = jnp.left_shift(1, lev - 1 - t)
            is_lane = s < _SC_
            lsh_f = jnp.where(is_lane, _SC_ - s, 0)
            rsh_f = jnp.where(is_lane, 0, _SR - lax.shift_right_logical(s, 7))
            lsh_b = jnp.where(is_lane, s, 0)
            rsh_b = jnp.where(is_lane, 0, lax.shift_right_logical(s, 7))
            lower = (n & s) == 0
            kp = jnp.where(lower, _partner(k, lsh_f, rsh_f),
                           _partner(k, lsh_b, rsh_b))
            ip = jnp.where(lower, _partner(i, lsh_f, rsh_f),
                           _partner(i, lsh_b, rsh_b))
            pred = (k > kp) | ((k == kp) & (i < ip))
            take = up == (lower == pred)
            return jnp.where(take, k, kp), jnp.where(take, i, ip)

        k, i = lax.fori_loop(0, lev, step, (k, i))
    ko_ref[0] = k
    io_ref[0] = i


def _topk_sort(scores):
    sp = jnp.concatenate(
        [scores, jnp.full((B, _NPAD - N), -1.0, jnp.float32)], axis=1
    ).reshape(B, _SR, _SC_)
    ks, idx = pl.pallas_call(
        _sort_body,
        grid=(B,),
        in_specs=[pl.BlockSpec((1, _SR, _SC_), lambda i: (i, 0, 0))],
        out_specs=[
            pl.BlockSpec((1, _SR, _SC_), lambda i: (i, 0, 0)),
            pl.BlockSpec((1, _SR, _SC_), lambda i: (i, 0, 0)),
        ],
        out_shape=[
            jax.ShapeDtypeStruct((B, _SR, _SC_), jnp.float32),
            jax.ShapeDtypeStruct((B, _SR, _SC_), jnp.int32),
        ],
    )(sp)
    return ks.reshape(B, _NPAD), idx.reshape(B, _NPAD)


_NW = 32
_KPAD = 25088
_RPW = _KPAD // _NW
_CH = 98
_NCH = _RPW // _CH


def _gather_body(x_hbm, idx_hbm, out_hbm, idx_v, rows_v, sem):
    wid = lax.axis_index("s") * 2 + lax.axis_index("c")
    for b in range(B):
        pltpu.sync_copy(idx_hbm.at[b, pl.ds(wid * _NCH, _NCH), :], idx_v)
        for c in range(_NCH):
            pltpu.async_copy(
                x_hbm.at[idx_v.at[c]],
                rows_v.at[pl.ds(c * _CH, _CH)],
                sem,
            )
        for c in range(_NCH):
            pltpu.make_async_copy(
                x_hbm.at[idx_v.at[c]],
                rows_v.at[pl.ds(c * _CH, _CH)],
                sem,
            ).wait()
        pltpu.sync_copy(rows_v, out_hbm.at[b, pl.ds(wid * _RPW, _RPW), :])


def _gather_rows(x, idx_pad):
    x2d = x.reshape(B * N, D)
    offs = (jnp.arange(B, dtype=jnp.int32) * N)[:, None]
    idx3 = (idx_pad + offs).reshape(B, _NW * _NCH, _CH)
    mesh = plsc.VectorSubcoreMesh(core_axis_name="c", subcore_axis_name="s")
    return pl.kernel(
        _gather_body,
        out_type=jax.ShapeDtypeStruct((B, _KPAD, D), jnp.float32),
        mesh=mesh,
        scratch_types=[
            pltpu.VMEM((_NCH, _CH), jnp.int32),
            pltpu.VMEM((_RPW, D), jnp.float32),
            pltpu.SemaphoreType.DMA,
        ],
    )(x2d, idx3)


_CBLK = 1000


def _scale_body(r_ref, v_ref, o_ref):
    o_ref[0] = r_ref[0] * v_ref[0, 0][:, None]


def _scale(raw_pad, vals):
    nblk = K // _CBLK
    grid = (B, nblk)
    vals3 = vals[:, :K].reshape(B * nblk, 1, _CBLK)
    return pl.pallas_call(
        _scale_body,
        grid=grid,
        in_specs=[
            pl.BlockSpec((1, _CBLK, D), lambda i, j: (i, j, 0)),
            pl.BlockSpec((1, 1, _CBLK), lambda i, j: (i * nblk + j, 0, 0)),
        ],
        out_specs=pl.BlockSpec((1, _CBLK, D), lambda i, j: (i, j, 0)),
        out_shape=jax.ShapeDtypeStruct((B, K, D), jnp.float32),
    )(raw_pad, vals3)


def kernel(x, W, b):
    scores = _scores(x, W, b)
    ks, idx = _topk_sort(scores)
    top_vals = ks[:, :K]
    idx_pad = jnp.minimum(idx[:, :_KPAD], N - 1)
    raw = _gather_rows(x, idx_pad)
    return _scale(raw, top_vals)


# speedup vs baseline: 1.3878x; 1.3878x over previous
import functools

import jax
import jax.numpy as jnp
from jax import lax
from jax.experimental import pallas as pl
from jax.experimental.pallas import tpu as pltpu
from jax.experimental.pallas import tpu_sc as plsc

B, N, D = 4, 50000, 128
K = 25000


_SBLK = 2000


def _score_body(x_ref, w_ref, b_ref, o_ref):
    xb = x_ref[0].astype(jnp.bfloat16)
    wb = w_ref[...].astype(jnp.bfloat16)
    logits = jnp.dot(xb, wb, preferred_element_type=jnp.float32)
    o_ref[0, 0, :] = jax.nn.sigmoid(logits[:, 0] + b_ref[0, 0])


def _scores(x, W, b):
    nblk = N // _SBLK
    grid = (B, nblk)
    out = pl.pallas_call(
        _score_body,
        grid=grid,
        in_specs=[
            pl.BlockSpec((1, _SBLK, D), lambda i, j: (i, j, 0)),
            pl.BlockSpec((D, D), lambda i, j: (0, 0)),
            pl.BlockSpec(memory_space=pltpu.SMEM),
        ],
        out_specs=pl.BlockSpec((1, 1, _SBLK), lambda i, j: (i * nblk + j, 0, 0)),
        out_shape=jax.ShapeDtypeStruct((B * nblk, 1, _SBLK), jnp.float32),
    )(x, jnp.zeros((D, D), jnp.float32).at[:, 0].set(W[0]), b.reshape(1, 1))
    return out.reshape(B, N)


_SR, _SC_ = 512, 128
_NPAD = _SR * _SC_


def _sort_body(k_ref, ko_ref, io_ref):
    k = k_ref[0]
    row = lax.broadcasted_iota(jnp.int32, (_SR, _SC_), 0)
    lane = lax.broadcasted_iota(jnp.int32, (_SR, _SC_), 1)
    n = row * _SC_ + lane
    i = n
    for lev in range(1, 17):
        kb = 1 << lev
        up = (n & kb) == 0

        def step(t, carry, off, axis, sz, sdiv, up=up):
            k, i = carry
            s = jnp.left_shift(1, off - t)
            sa = lax.shift_right_logical(s, sdiv)
            lower = (n & s) == 0
            kp = jnp.where(lower, pltpu.roll(k, sz - sa, axis),
                           pltpu.roll(k, sa, axis))
            ip = jnp.where(lower, pltpu.roll(i, sz - sa, axis),
                           pltpu.roll(i, sa, axis))
            pred = (k > kp) | ((k == kp) & (i < ip))
            take = up == (lower == pred)
            return jnp.where(take, k, kp), jnp.where(take, i, ip)

        if lev > 7:
            row_step = functools.partial(step, off=lev - 1, axis=0,
                                         sz=_SR, sdiv=7)
            k, i = lax.fori_loop(0, lev - 7, row_step, (k, i))
        lane_hi = min(lev, 7)
        lane_step = functools.partial(step, off=lane_hi - 1, axis=1,
                                      sz=_SC_, sdiv=0)
        k, i = lax.fori_loop(0, lane_hi, lane_step, (k, i))
    ko_ref[0] = k
    io_ref[0] = i


def _topk_sort(scores):
    sp = jnp.concatenate(
        [scores, jnp.full((B, _NPAD - N), -1.0, jnp.float32)], axis=1
    ).reshape(B, _SR, _SC_)
    ks, idx = pl.pallas_call(
        _sort_body,
        grid=(B,),
        in_specs=[pl.BlockSpec((1, _SR, _SC_), lambda i: (i, 0, 0))],
        out_specs=[
            pl.BlockSpec((1, _SR, _SC_), lambda i: (i, 0, 0)),
            pl.BlockSpec((1, _SR, _SC_), lambda i: (i, 0, 0)),
        ],
        out_shape=[
            jax.ShapeDtypeStruct((B, _SR, _SC_), jnp.float32),
            jax.ShapeDtypeStruct((B, _SR, _SC_), jnp.int32),
        ],
    )(sp)
    return ks.reshape(B, _NPAD), idx.reshape(B, _NPAD)


_NW = 32
_KPAD = 25088
_RPW = _KPAD // _NW
_CH = 98
_NCH = _RPW // _CH


def _gather_body(x_hbm, idx_hbm, out_hbm, idx_v, rows_v, sem):
    wid = lax.axis_index("s") * 2 + lax.axis_index("c")
    for b in range(B):
        pltpu.sync_copy(idx_hbm.at[b, pl.ds(wid * _NCH, _NCH), :], idx_v)
        for c in range(_NCH):
            pltpu.async_copy(
                x_hbm.at[idx_v.at[c]],
                rows_v.at[pl.ds(c * _CH, _CH)],
                sem,
            )
        for c in range(_NCH):
            pltpu.make_async_copy(
                x_hbm.at[idx_v.at[c]],
                rows_v.at[pl.ds(c * _CH, _CH)],
                sem,
            ).wait()
        pltpu.sync_copy(rows_v, out_hbm.at[b, pl.ds(wid * _RPW, _RPW), :])


def _gather_rows(x, idx_pad):
    x2d = x.reshape(B * N, D)
    offs = (jnp.arange(B, dtype=jnp.int32) * N)[:, None]
    idx3 = (idx_pad + offs).reshape(B, _NW * _NCH, _CH)
    mesh = plsc.VectorSubcoreMesh(core_axis_name="c", subcore_axis_name="s")
    return pl.kernel(
        _gather_body,
        out_type=jax.ShapeDtypeStruct((B, _KPAD, D), jnp.float32),
        mesh=mesh,
        scratch_types=[
            pltpu.VMEM((_NCH, _CH), jnp.int32),
            pltpu.VMEM((_RPW, D), jnp.float32),
            pltpu.SemaphoreType.DMA,
        ],
    )(x2d, idx3)


_CBLK = 1000


def _scale_body(r_ref, v_ref, o_ref):
    o_ref[0] = r_ref[0] * v_ref[0, 0][:, None]


def _scale(raw_pad, vals):
    nblk = K // _CBLK
    grid = (B, nblk)
    vals3 = vals[:, :K].reshape(B * nblk, 1, _CBLK)
    return pl.pallas_call(
        _scale_body,
        grid=grid,
        in_specs=[
            pl.BlockSpec((1, _CBLK, D), lambda i, j: (i, j, 0)),
            pl.BlockSpec((1, 1, _CBLK), lambda i, j: (i * nblk + j, 0, 0)),
        ],
        out_specs=pl.BlockSpec((1, _CBLK, D), lambda i, j: (i, j, 0)),
        out_shape=jax.ShapeDtypeStruct((B, K, D), jnp.float32),
    )(raw_pad, vals3)


def kernel(x, W, b):
    scores = _scores(x, W, b)
    ks, idx = _topk_sort(scores)
    top_vals = ks[:, :K]
    idx_pad = jnp.minimum(idx[:, :_KPAD], N - 1)
    raw = _gather_rows(x, idx_pad)
    return _scale(raw, top_vals)
